# baseline (speedup 1.0000x reference)
import jax
import jax.numpy as jnp
from jax.experimental import pallas as pl
from jax.experimental.pallas import tpu as pltpu

_B, _C, _H, _W = 8, 1, 512, 512
_N = _B * _C * _H * _W
_ROWS = 2048
_COLS = 1024
_BLK_ROWS = 1024


def _xent_sum_kernel(o_ref, l_ref, acc_ref):
    o = o_ref[...]
    lab = (l_ref[...] >= 0.5).astype(jnp.float32)
    gtz = (o >= 0).astype(jnp.float32)
    loss = o * (lab - gtz) - jnp.log(1.0 + jnp.exp(o - 2.0 * o * gtz))
    s = jnp.sum(-loss).reshape(1, 1)

    @pl.when(pl.program_id(0) == 0)
    def _init():
        acc_ref[...] = jnp.zeros((1, 1), jnp.float32)

    acc_ref[...] = acc_ref[...] + s

    @pl.when(pl.program_id(0) == pl.num_programs(0) - 1)
    def _final():
        acc_ref[...] = acc_ref[...] * jnp.float32(1.0 / _N)


def kernel(output, label):
    o2 = output.reshape(_ROWS, _COLS)
    l2 = label.reshape(_ROWS, _COLS)
    grid = _ROWS // _BLK_ROWS
    acc = pl.pallas_call(
        _xent_sum_kernel,
        grid=(grid,),
        in_specs=[
            pl.BlockSpec((_BLK_ROWS, _COLS), lambda i: (i, 0)),
            pl.BlockSpec((_BLK_ROWS, _COLS), lambda i: (i, 0)),
        ],
        out_specs=pl.BlockSpec((1, 1), lambda i: (0, 0)),
        out_shape=jax.ShapeDtypeStruct((1, 1), jnp.float32),
    )(o2, l2)
    return acc[0, 0]


# speedup vs baseline: 3.0744x; 3.0744x over previous
import jax
import jax.numpy as jnp
from jax.experimental import pallas as pl
from jax.experimental.pallas import tpu as pltpu

_B, _C, _H, _W = 8, 1, 512, 512
_N = _B * _C * _H * _W
_BLK_B = 4


def _xent_sum_kernel(o_ref, l_ref, acc_ref):
    o = o_ref[...]
    lab = (l_ref[...] >= 0.5).astype(jnp.float32)
    gtz = (o >= 0).astype(jnp.float32)
    loss = o * (lab - gtz) - jnp.log(1.0 + jnp.exp(o - 2.0 * o * gtz))
    s = jnp.sum(-loss).reshape(1, 1)

    @pl.when(pl.program_id(0) == 0)
    def _init():
        acc_ref[...] = jnp.zeros((1, 1), jnp.float32)

    acc_ref[...] = acc_ref[...] + s

    @pl.when(pl.program_id(0) == pl.num_programs(0) - 1)
    def _final():
        acc_ref[...] = acc_ref[...] * jnp.float32(1.0 / _N)


def kernel(output, label):
    grid = _B // _BLK_B
    acc = pl.pallas_call(
        _xent_sum_kernel,
        grid=(grid,),
        in_specs=[
            pl.BlockSpec((_BLK_B, _C, _H, _W), lambda i: (i, 0, 0, 0)),
            pl.BlockSpec((_BLK_B, _C, _H, _W), lambda i: (i, 0, 0, 0)),
        ],
        out_specs=pl.BlockSpec((1, 1), lambda i: (0, 0)),
        out_shape=jax.ShapeDtypeStruct((1, 1), jnp.float32),
    )(output, label)
    return acc[0, 0]


# speedup vs baseline: 3.1202x; 1.0149x over previous
import jax
import jax.numpy as jnp
from jax.experimental import pallas as pl
from jax.experimental.pallas import tpu as pltpu

_B, _C, _H, _W = 8, 1, 512, 512
_N = _B * _C * _H * _W
_BLK_B = 2


def _xent_sum_kernel(o_ref, l_ref, acc_ref):
    o = o_ref[...]
    lab = (l_ref[...] >= 0.5).astype(jnp.float32)
    gtz = (o >= 0).astype(jnp.float32)
    loss = o * (lab - gtz) - jnp.log(1.0 + jnp.exp(o - 2.0 * o * gtz))
    s = jnp.sum(-loss).reshape(1, 1)

    @pl.when(pl.program_id(0) == 0)
    def _init():
        acc_ref[...] = jnp.zeros((1, 1), jnp.float32)

    acc_ref[...] = acc_ref[...] + s

    @pl.when(pl.program_id(0) == pl.num_programs(0) - 1)
    def _final():
        acc_ref[...] = acc_ref[...] * jnp.float32(1.0 / _N)


def kernel(output, label):
    grid = _B // _BLK_B
    acc = pl.pallas_call(
        _xent_sum_kernel,
        grid=(grid,),
        in_specs=[
            pl.BlockSpec((_BLK_B, _C, _H, _W), lambda i: (i, 0, 0, 0)),
            pl.BlockSpec((_BLK_B, _C, _H, _W), lambda i: (i, 0, 0, 0)),
        ],
        out_specs=pl.BlockSpec((1, 1), lambda i: (0, 0)),
        out_shape=jax.ShapeDtypeStruct((1, 1), jnp.float32),
    )(output, label)
    return acc[0, 0]
